# baseline (speedup 1.0000x reference)
import jax
import jax.numpy as jnp
from jax import lax
from jax.experimental import pallas as pl

B, T, E, C = 4, 2048, 8, 512
IN_F = 2048
OUT_F = 2048
OE = OUT_F // E
EC = E * C
TT = 256
NT = T // TT
INV_SQRT2 = 0.7071067811865476


def _prep_body(w1_ref, w2_ref, b2_ref, w1s_ref, w2s_ref, bsum_ref):
    e = pl.program_id(0)
    w1s_ref[0, 0, :] = jnp.sum(w1_ref[...], axis=1)
    acc = jnp.zeros((OE,), jnp.float32)
    for k in range(IN_F // OE):
        acc = acc + jnp.sum(w2_ref[:, k * OE:(k + 1) * OE], axis=0)
    w2s_ref[0, 0, :] = acc

    @pl.when(e == 0)
    def _():
        bsum_ref[0, 0, :] = jnp.full((128,), jnp.sum(b2_ref[...]), jnp.float32)


def _s_body(x_ref, dm_ref, s_ref):
    t = pl.program_id(1)

    @pl.when(t == 0)
    def _():
        s_ref[...] = jnp.zeros_like(s_ref)

    xs_t = jnp.sum(x_ref[0], axis=-1)
    s_ref[0, 0, :] += jnp.sum(dm_ref[0] * xs_t[:, None], axis=0)


def _g_body(s_ref, w1s_ref, b1_ref, w2s_ref, bsum_ref, g_ref):
    sv = s_ref[0, 0, :]
    a = sv[:, None] * w1s_ref[0, 0][None, :] + b1_ref[0, 0][None, :]
    h = 0.5 * a * (1.0 + lax.erf(a * INV_SQRT2))
    g_ref[0, 0, :] = jnp.sum(h * w2s_ref[0, 0][None, :], axis=1) + bsum_ref[0, 0, :1]


def _combine_body(cm_ref, g_ref, out_ref):
    out_ref[0, 0, :] = jnp.sum(cm_ref[0] * g_ref[0], axis=-1)


def kernel(x, dispatch_mask, combine_array, W1, b1, W2, b2):
    dm3 = dispatch_mask.reshape(B, T, EC)
    cm3 = combine_array.reshape(B, T, EC)
    b1r = b1.reshape(E, 1, OE)
    b2r = b2.reshape(1, OUT_F)

    w1s, w2s, bsum = pl.pallas_call(
        _prep_body,
        grid=(E,),
        in_specs=[
            pl.BlockSpec((OE, IN_F), lambda e: (e, 0)),
            pl.BlockSpec((OE, IN_F), lambda e: (e, 0)),
            pl.BlockSpec((1, OUT_F), lambda e: (0, 0)),
        ],
        out_specs=[
            pl.BlockSpec((1, 1, OE), lambda e: (e, 0, 0)),
            pl.BlockSpec((1, 1, OE), lambda e: (e, 0, 0)),
            pl.BlockSpec((1, 1, 128), lambda e: (0, 0, 0)),
        ],
        out_shape=[
            jax.ShapeDtypeStruct((E, 1, OE), jnp.float32),
            jax.ShapeDtypeStruct((E, 1, OE), jnp.float32),
            jax.ShapeDtypeStruct((1, 1, 128), jnp.float32),
        ],
    )(W1, W2, b2r)

    s = pl.pallas_call(
        _s_body,
        grid=(B, NT),
        in_specs=[
            pl.BlockSpec((1, TT, IN_F), lambda b, t: (b, t, 0)),
            pl.BlockSpec((1, TT, EC), lambda b, t: (b, t, 0)),
        ],
        out_specs=pl.BlockSpec((1, 1, EC), lambda b, t: (b, 0, 0)),
        out_shape=jax.ShapeDtypeStruct((B, 1, EC), jnp.float32),
    )(x, dm3)

    g = pl.pallas_call(
        _g_body,
        grid=(B, E),
        in_specs=[
            pl.BlockSpec((1, 1, C), lambda b, e: (b, 0, e)),
            pl.BlockSpec((1, 1, OE), lambda b, e: (e, 0, 0)),
            pl.BlockSpec((1, 1, OE), lambda b, e: (e, 0, 0)),
            pl.BlockSpec((1, 1, OE), lambda b, e: (e, 0, 0)),
            pl.BlockSpec((1, 1, 128), lambda b, e: (0, 0, 0)),
        ],
        out_specs=pl.BlockSpec((1, 1, C), lambda b, e: (b, 0, e)),
        out_shape=jax.ShapeDtypeStruct((B, 1, EC), jnp.float32),
    )(s, w1s, b1r, w2s, bsum)

    out = pl.pallas_call(
        _combine_body,
        grid=(B, NT),
        in_specs=[
            pl.BlockSpec((1, TT, EC), lambda b, t: (b, t, 0)),
            pl.BlockSpec((1, 1, EC), lambda b, t: (b, 0, 0)),
        ],
        out_specs=pl.BlockSpec((1, 1, TT), lambda b, t: (b * NT + t, 0, 0)),
        out_shape=jax.ShapeDtypeStruct((B * NT, 1, TT), jnp.float32),
    )(cm3, g)

    return out.reshape(B, T)
